# baseline (speedup 1.0000x reference)
import jax
import jax.numpy as jnp
from jax import lax
from jax.experimental import pallas as pl
from jax.experimental.pallas import tpu as pltpu
from jax.experimental.pallas import tpu_sc as plsc

B = 256
N = 100000
G = 5000
HALF_G = G // 2

NC = 2
NS = 16
NW = NC * NS
R = B // NW
TW = 128
NT = 11
NB = NT * TW
NUM_BLOCKS = 71
NMAIN = NUM_BLOCKS * NB
L = 16
HP = TW // 2
UNROLL = 8


def _sc_kernel(x_hbm, seg_hbm, w_hbm, xt_hbm, st_hbm, wt_hbm, out_hbm,
               x_buf, seg_buf, w_buf, acc, sem):
    wid = lax.axis_index("s") * NC + lax.axis_index("c")
    row0 = wid * R

    lane = lax.iota(jnp.int32, L)
    lane_r = lane & (R - 1)
    hi = lane >= R
    hi64 = jnp.where(hi, HP, 0).astype(jnp.int32)
    sbase = lane_r * G
    zeros = jnp.zeros((L,), jnp.float32)

    def issue(blk, par):
        off = blk * NB
        pltpu.async_copy(seg_hbm.at[pl.ds(off, NB)],
                         seg_buf.at[pl.ds(par * NB, NB)], sem)
        pltpu.async_copy(w_hbm.at[pl.ds(off, NB)],
                         w_buf.at[pl.ds(par * NB, NB)], sem)
        for t in range(NT):
            pltpu.async_copy(
                x_hbm.at[pl.ds(row0, R), pl.ds(off + t * TW, TW)],
                x_buf.at[par * NT + t], sem)

    def drain():
        pltpu.make_async_copy(seg_hbm.at[pl.ds(0, NB)],
                              seg_buf.at[pl.ds(0, NB)], sem).wait()
        pltpu.make_async_copy(w_hbm.at[pl.ds(0, NB)],
                              w_buf.at[pl.ds(0, NB)], sem).wait()
        for t in range(NT):
            pltpu.make_async_copy(x_hbm.at[pl.ds(0, R), pl.ds(0, TW)],
                                  x_buf.at[0], sem).wait()

    @plsc.parallel_loop(0, R * G // L, unroll=8)
    def zero_body(i):
        acc[pl.ds(i * L, L)] = zeros

    def tile_compute(tv, soff):
        ghoist = hi64 + soff

        @plsc.parallel_loop(0, HP, unroll=UNROLL)
        def pair_body(c):
            cv = hi64 + c
            gs = ghoist + c
            xv = plsc.load_gather(x_buf, [tv, lane_r, cv])
            sv = plsc.load_gather(seg_buf, [gs])
            wv = plsc.load_gather(w_buf, [gs])
            wfv = jnp.where(sv >= HALF_G, wv, jnp.float32(1.0))
            plsc.addupdate_scatter(acc, [sbase + sv], xv * wfv)

    issue(0, 0)

    def block_body(blk, carry):
        par = blk & 1
        drain()

        @pl.when(blk + 1 < NUM_BLOCKS)
        def _():
            issue(blk + 1, 1 - par)

        def t_body(t, c2):
            tile_compute(jnp.full((L,), par * NT, jnp.int32) + t,
                         par * NB + t * TW)
            return c2

        lax.fori_loop(0, NT, t_body, 0)
        return carry

    lax.fori_loop(0, NUM_BLOCKS, block_body, 0)

    pltpu.sync_copy(xt_hbm.at[pl.ds(row0, R), :], x_buf.at[0])
    pltpu.sync_copy(st_hbm, seg_buf.at[pl.ds(0, TW)])
    pltpu.sync_copy(wt_hbm, w_buf.at[pl.ds(0, TW)])
    tile_compute(jnp.zeros((L,), jnp.int32), 0)

    pltpu.sync_copy(acc, out_hbm.at[pl.ds(row0 * G, R * G)])


def kernel(x, segment_ids, W):
    xt = jnp.pad(x[:, NMAIN:], ((0, 0), (0, TW - (N - NMAIN))))
    st = jnp.pad(segment_ids[NMAIN:], (0, TW - (N - NMAIN)))
    wt = jnp.pad(W[NMAIN:], (0, TW - (N - NMAIN)))
    mesh = plsc.VectorSubcoreMesh(core_axis_name="c", subcore_axis_name="s")
    f = pl.kernel(
        _sc_kernel,
        mesh=mesh,
        compiler_params=pltpu.CompilerParams(needs_layout_passes=False),
        out_type=jax.ShapeDtypeStruct((B * G,), jnp.float32),
        scratch_types=[
            pltpu.VMEM((2 * NT, R, TW), jnp.float32),
            pltpu.VMEM((2 * NB,), jnp.int32),
            pltpu.VMEM((2 * NB,), jnp.float32),
            pltpu.VMEM((R * G,), jnp.float32),
            pltpu.SemaphoreType.DMA,
        ],
    )
    return f(x, segment_ids, W, xt, st, wt).reshape(B, G)


# speedup vs baseline: 1.3009x; 1.3009x over previous
import jax
import jax.numpy as jnp
from jax import lax
from jax.experimental import pallas as pl
from jax.experimental.pallas import tpu as pltpu
from jax.experimental.pallas import tpu_sc as plsc

B = 256
N = 100000
G = 5000
HALF_G = G // 2

NC = 2
NS = 16
NW = NC * NS
R = B // NW
TW = 128
NT = 11
NB = NT * TW
NBP = NB + 8
NUM_BLOCKS = 71
NMAIN = NUM_BLOCKS * NB
L = 16
H = NB // 2
UNROLL = 8


def _sc_kernel(x_hbm, seg_hbm, w_hbm, xt_hbm, st_hbm, wt_hbm, out_hbm,
               x_buf, x_lin, seg_buf, w_buf, acc, sem):
    wid = lax.axis_index("s") * NC + lax.axis_index("c")
    row0 = wid * R

    lane = lax.iota(jnp.int32, L)
    lane_r = lane & (R - 1)
    hi = lane >= R
    sbase = lane_r * G
    rbase = lane_r * NBP
    zeros = jnp.zeros((L,), jnp.float32)

    def issue(blk, par):
        off = blk * NB
        pltpu.async_copy(seg_hbm.at[pl.ds(off, NB)],
                         seg_buf.at[pl.ds(par * NB, NB)], sem)
        pltpu.async_copy(w_hbm.at[pl.ds(off, NB)],
                         w_buf.at[pl.ds(par * NB, NB)], sem)
        pltpu.async_copy(x_hbm.at[pl.ds(row0, R), pl.ds(off, NB)],
                         x_buf.at[:, pl.ds(par * NB, NB)], sem)

    def drain():
        pltpu.make_async_copy(seg_hbm.at[pl.ds(0, NB)],
                              seg_buf.at[pl.ds(0, NB)], sem).wait()
        pltpu.make_async_copy(w_hbm.at[pl.ds(0, NB)],
                              w_buf.at[pl.ds(0, NB)], sem).wait()
        pltpu.make_async_copy(x_hbm.at[pl.ds(0, R), pl.ds(0, NB)],
                              x_buf.at[:, pl.ds(0, NB)], sem).wait()

    @plsc.parallel_loop(0, R * G // L, unroll=8)
    def zero_body(i):
        acc[pl.ds(i * L, L)] = zeros

    def relayout(par, ncols):
        for r in range(R):
            @plsc.parallel_loop(0, ncols // L, unroll=8)
            def copy_body(i):
                v = x_buf[r, pl.ds(par * NB + i * L, L)]
                x_lin[pl.ds(r * NBP + i * L, L)] = v

    def block_compute(npairs, soff, hvec):
        xhoist = rbase + hvec
        shoist = hvec + soff

        @plsc.parallel_loop(0, npairs, unroll=UNROLL)
        def pair_body(j):
            gs = shoist + j
            xv = plsc.load_gather(x_lin, [xhoist + j])
            sv = plsc.load_gather(seg_buf, [gs])
            wv = plsc.load_gather(w_buf, [gs])
            wfv = jnp.where(sv >= HALF_G, wv, jnp.float32(1.0))
            plsc.addupdate_scatter(acc, [sbase + sv], xv * wfv)

    hi_h = jnp.where(hi, H, 0).astype(jnp.int32)
    hi_t = jnp.where(hi, TW // 2, 0).astype(jnp.int32)

    issue(0, 0)

    def block_body(blk, carry):
        par = blk & 1
        drain()

        @pl.when(blk + 1 < NUM_BLOCKS)
        def _():
            issue(blk + 1, 1 - par)

        relayout(par, NB)
        block_compute(H, par * NB, hi_h)
        return carry

    lax.fori_loop(0, NUM_BLOCKS, block_body, 0)

    pltpu.sync_copy(xt_hbm.at[pl.ds(row0, R), :], x_buf.at[:, pl.ds(0, TW)])
    pltpu.sync_copy(st_hbm, seg_buf.at[pl.ds(0, TW)])
    pltpu.sync_copy(wt_hbm, w_buf.at[pl.ds(0, TW)])
    relayout(0, TW)
    block_compute(TW // 2, 0, hi_t)

    pltpu.sync_copy(acc, out_hbm.at[pl.ds(row0 * G, R * G)])


def kernel(x, segment_ids, W):
    xt = jnp.pad(x[:, NMAIN:], ((0, 0), (0, TW - (N - NMAIN))))
    st = jnp.pad(segment_ids[NMAIN:], (0, TW - (N - NMAIN)))
    wt = jnp.pad(W[NMAIN:], (0, TW - (N - NMAIN)))
    mesh = plsc.VectorSubcoreMesh(core_axis_name="c", subcore_axis_name="s")
    f = pl.kernel(
        _sc_kernel,
        mesh=mesh,
        compiler_params=pltpu.CompilerParams(needs_layout_passes=False),
        out_type=jax.ShapeDtypeStruct((B * G,), jnp.float32),
        scratch_types=[
            pltpu.VMEM((R, 2 * NB), jnp.float32),
            pltpu.VMEM((R * NBP,), jnp.float32),
            pltpu.VMEM((2 * NB,), jnp.int32),
            pltpu.VMEM((2 * NB,), jnp.float32),
            pltpu.VMEM((R * G,), jnp.float32),
            pltpu.SemaphoreType.DMA,
        ],
    )
    return f(x, segment_ids, W, xt, st, wt).reshape(B, G)


# speedup vs baseline: 3.4565x; 2.6569x over previous
import jax
import jax.numpy as jnp
from jax import lax
from jax.experimental import pallas as pl
from jax.experimental.pallas import tpu as pltpu
from jax.experimental.pallas import tpu_sc as plsc

B = 256
N = 100000
G = 5000
HALF_G = G // 2

NC = 2
NS = 16
NW = NC * NS
R = B // NW
NB = 2000
NUM_BLOCKS = N // NB
L = 16
H = NB // 2
UNROLL = 8


def _sc_kernel(x_hbm, seg_hbm, w_hbm, out_hbm,
               x_buf, seg_buf, w_buf, acc, sem):
    wid = lax.axis_index("s") * NC + lax.axis_index("c")
    row0 = wid * R

    lane = lax.iota(jnp.int32, L)
    lane_r = lane & (R - 1)
    hi = lane >= R
    hi_off = jnp.where(hi, H, 0).astype(jnp.int32)
    gbase = lane_r * NB + hi_off
    sbase = lane_r * G
    zeros = jnp.zeros((L,), jnp.float32)

    def issue(blk, par):
        off = blk * NB
        pltpu.async_copy(seg_hbm.at[pl.ds(off, NB)],
                         seg_buf.at[pl.ds(par * NB, NB)], sem)
        pltpu.async_copy(w_hbm.at[pl.ds(off, NB)],
                         w_buf.at[pl.ds(par * NB, NB)], sem)
        for r in range(R):
            pltpu.async_copy(
                x_hbm.at[pl.ds((row0 + r) * N + off, NB)],
                x_buf.at[pl.ds((par * R + r) * NB, NB)], sem)

    def drain():
        pltpu.make_async_copy(seg_hbm.at[pl.ds(0, NB)],
                              seg_buf.at[pl.ds(0, NB)], sem).wait()
        pltpu.make_async_copy(w_hbm.at[pl.ds(0, NB)],
                              w_buf.at[pl.ds(0, NB)], sem).wait()
        for r in range(R):
            pltpu.make_async_copy(x_hbm.at[pl.ds(0, NB)],
                                  x_buf.at[pl.ds(0, NB)], sem).wait()

    @plsc.parallel_loop(0, R * G // L, unroll=8)
    def zero_body(i):
        acc[pl.ds(i * L, L)] = zeros

    issue(0, 0)

    def block_body(blk, carry):
        par = blk & 1
        drain()

        @pl.when(blk + 1 < NUM_BLOCKS)
        def _():
            issue(blk + 1, 1 - par)

        xoff = par * (R * NB)
        soff = par * NB

        sv_last = seg_buf[pl.ds(soff + NB - L, L)]
        unweighted = sv_last[L - 1] < HALF_G

        @pl.when(unweighted)
        def _():
            @plsc.parallel_loop(0, H, unroll=UNROLL)
            def fast_body(j):
                xv = plsc.load_gather(x_buf, [gbase + (j + xoff)])
                sv = plsc.load_gather(seg_buf, [hi_off + (j + soff)])
                plsc.addupdate_scatter(acc, [sbase + sv], xv)

        @pl.when(jnp.logical_not(unweighted))
        def _():
            @plsc.parallel_loop(0, H, unroll=UNROLL)
            def pair_body(j):
                gs = hi_off + (j + soff)
                xv = plsc.load_gather(x_buf, [gbase + (j + xoff)])
                sv = plsc.load_gather(seg_buf, [gs])
                wv = plsc.load_gather(w_buf, [gs])
                wfv = jnp.where(sv >= HALF_G, wv, jnp.float32(1.0))
                plsc.addupdate_scatter(acc, [sbase + sv], xv * wfv)

        return carry

    lax.fori_loop(0, NUM_BLOCKS, block_body, 0)

    pltpu.sync_copy(acc, out_hbm.at[pl.ds(row0 * G, R * G)])


def kernel(x, segment_ids, W):
    mesh = plsc.VectorSubcoreMesh(core_axis_name="c", subcore_axis_name="s")
    f = pl.kernel(
        _sc_kernel,
        mesh=mesh,
        compiler_params=pltpu.CompilerParams(
            needs_layout_passes=False, use_tc_tiling_on_sc=False),
        out_type=jax.ShapeDtypeStruct((B * G,), jnp.float32),
        scratch_types=[
            pltpu.VMEM((2 * R * NB,), jnp.float32),
            pltpu.VMEM((2 * NB,), jnp.int32),
            pltpu.VMEM((2 * NB,), jnp.float32),
            pltpu.VMEM((R * G,), jnp.float32),
            pltpu.SemaphoreType.DMA,
        ],
    )
    return f(x.reshape(-1), segment_ids, W).reshape(B, G)
